# baseline (speedup 1.0000x reference)
import functools

import jax
import jax.numpy as jnp
from jax import lax
from jax.experimental import pallas as pl
from jax.experimental.pallas import tpu as pltpu
from jax.experimental.pallas import tpu_sc as plsc

N_NODES = 10000
N_EDGES = 320000
DIM = 128
HID = 128
N_LAYERS = 4
N_PROPS = 4
N_GRAPHS = 128

NC = 2
NS = 16
NW = NC * NS
CW = 128
NCHUNK = 80
EPW = NCHUNK * CW
EPAD = NW * EPW
NPAD = 10240
RPT = NPAD // NS

_HIGH = lax.Precision.HIGHEST
_TC_PARAMS = pltpu.CompilerParams(vmem_limit_bytes=120 * 1024 * 1024)
_mesh = plsc.VectorSubcoreMesh(core_axis_name="c", subcore_axis_name="s")


@functools.partial(
    pl.kernel,
    mesh=_mesh,
    out_type=jax.ShapeDtypeStruct((NC, NS, RPT), jnp.float32),
    scratch_types=[
        pltpu.VMEM((NCHUNK, CW), jnp.int32),
        pltpu.VMEM((CW,), jnp.float32),
        pltpu.VMEM((RPT,), jnp.float32),
        pltpu.VMEM_SHARED((NPAD,), jnp.float32),
    ],
)
def _sc_degree(dst_hbm, out_hbm, dst_v, ones_v, zrow_v, deg_sh):
    c = lax.axis_index("c")
    s = lax.axis_index("s")
    wid = s * NC + c
    zeros16 = jnp.zeros((16,), jnp.float32)
    ones16 = jnp.ones((16,), jnp.float32)
    for cc in range(CW // 16):
        ones_v[pl.ds(cc * 16, 16)] = ones16

    def zero_local(i, carry):
        zrow_v[pl.ds(i * 16, 16)] = zeros16
        return carry

    lax.fori_loop(0, RPT // 16, zero_local, 0)
    pltpu.sync_copy(zrow_v, deg_sh.at[pl.ds(s * RPT, RPT)])
    pltpu.sync_copy(dst_hbm.at[wid], dst_v)
    plsc.subcore_barrier()

    def count(j, carry):
        pltpu.sync_copy(ones_v, deg_sh.at[dst_v.at[j]], add=True)
        return carry

    lax.fori_loop(0, NCHUNK, count, 0)
    plsc.subcore_barrier()
    pltpu.sync_copy(deg_sh.at[pl.ds(s * RPT, RPT)], out_hbm.at[c, s])


@functools.partial(
    pl.kernel,
    mesh=_mesh,
    out_type=jax.ShapeDtypeStruct((NC, NPAD, HID), jnp.float32),
    scratch_types=[
        pltpu.VMEM((NCHUNK // 2, CW), jnp.int32),
        pltpu.VMEM((NCHUNK // 2, CW), jnp.int32),
        pltpu.VMEM((2, CW, HID), jnp.float32),
        pltpu.VMEM((16, HID), jnp.float32),
        pltpu.VMEM_SHARED((NPAD, HID), jnp.float32),
        pltpu.SemaphoreType.DMA,
        pltpu.SemaphoreType.DMA,
        pltpu.SemaphoreType.DMA,
        pltpu.SemaphoreType.DMA,
    ],
)
def _sc_aggregate(xs_hbm, src_hbm, dst_hbm, out_hbm,
                  src_v, dst_v, rows_v, zbuf, acc_sh,
                  gsem0, gsem1, ssem0, ssem1):
    c = lax.axis_index("c")
    s = lax.axis_index("s")
    wid = s * NC + c
    zeros16 = jnp.zeros((16,), jnp.float32)
    seg_n = NCHUNK // 2

    def zb_row(r, carry):
        for cc in range(HID // 16):
            zbuf[r, pl.ds(cc * 16, 16)] = zeros16
        return carry

    lax.fori_loop(0, 16, zb_row, 0)

    for seg in range(2):
        pltpu.sync_copy(src_hbm.at[wid, pl.ds(seg * seg_n, seg_n)], src_v)
        pltpu.sync_copy(dst_hbm.at[wid, pl.ds(seg * seg_n, seg_n)], dst_v)
        pltpu.async_copy(xs_hbm.at[src_v.at[0]], rows_v.at[0], gsem0)
        pltpu.async_copy(xs_hbm.at[src_v.at[1]], rows_v.at[1], gsem1)
        if seg == 0:
            def zero_slice(i, carry):
                pltpu.sync_copy(zbuf, acc_sh.at[pl.ds(s * RPT + i * 16, 16)])
                return carry

            lax.fori_loop(0, RPT // 16, zero_slice, 0)
            plsc.subcore_barrier()

        def pair(jj, carry):
            j0 = jj * 2
            pltpu.make_async_copy(xs_hbm.at[src_v.at[j0]],
                                  rows_v.at[0], gsem0).wait()
            pltpu.async_copy(rows_v.at[0], acc_sh.at[dst_v.at[j0]], ssem0,
                             add=True)
            pltpu.make_async_copy(xs_hbm.at[src_v.at[j0 + 1]],
                                  rows_v.at[1], gsem1).wait()
            pltpu.async_copy(rows_v.at[1], acc_sh.at[dst_v.at[j0 + 1]], ssem1,
                             add=True)
            pltpu.make_async_copy(rows_v.at[0], acc_sh.at[dst_v.at[j0]],
                                  ssem0).wait()

            @pl.when(j0 + 2 < seg_n)
            def _():
                pltpu.async_copy(xs_hbm.at[src_v.at[j0 + 2]],
                                 rows_v.at[0], gsem0)

            pltpu.make_async_copy(rows_v.at[1], acc_sh.at[dst_v.at[j0 + 1]],
                                  ssem1).wait()

            @pl.when(j0 + 3 < seg_n)
            def _():
                pltpu.async_copy(xs_hbm.at[src_v.at[j0 + 3]],
                                 rows_v.at[1], gsem1)

            return carry

        lax.fori_loop(0, seg_n // 2, pair, 0)

    plsc.subcore_barrier()
    pltpu.sync_copy(acc_sh.at[pl.ds(s * RPT, RPT)],
                    out_hbm.at[c, pl.ds(s * RPT, RPT)])


def _row_mask():
    rowid = lax.broadcasted_iota(jnp.int32, (NPAD, 1), 0)
    return (rowid < N_NODES).astype(jnp.float32)


def _tc_embed_body(x_ref, ew_ref, eb_ref, degp_ref, w0_ref,
                   h_ref, xs_ref, dis_ref):
    h = jnp.maximum(
        jnp.dot(x_ref[...], ew_ref[...], precision=_HIGH,
                preferred_element_type=jnp.float32) + eb_ref[...], 0.0)
    deg = degp_ref[0] + degp_ref[1] + 1.0
    dis = lax.rsqrt(deg) * _row_mask()
    h_ref[...] = h
    dis_ref[...] = dis
    xs_ref[...] = jnp.dot(h, w0_ref[...], precision=_HIGH,
                          preferred_element_type=jnp.float32) * dis


_tc_embed = pl.pallas_call(
    _tc_embed_body,
    out_shape=(
        jax.ShapeDtypeStruct((NPAD, HID), jnp.float32),
        jax.ShapeDtypeStruct((NPAD, HID), jnp.float32),
        jax.ShapeDtypeStruct((NPAD, 1), jnp.float32),
    ),
    compiler_params=_TC_PARAMS,
)


def _bn_block(sc_ref, xs_ref, h_ref, dis_ref, b_ref, gam_ref, bet_ref):
    mask = _row_mask()
    agg = ((sc_ref[0] + sc_ref[1] + xs_ref[...]) * dis_ref[...]
           + b_ref[...]) * mask
    mean = jnp.sum(agg, axis=0, keepdims=True) * (1.0 / N_NODES)
    d = (agg - mean) * mask
    var = jnp.sum(d * d, axis=0, keepdims=True) * (1.0 / N_NODES)
    hbn = (agg - mean) * lax.rsqrt(var + 1e-5) * gam_ref[...] + bet_ref[...]
    return jnp.maximum(hbn, 0.0) + h_ref[...]


def _tc_layer_body(sc_ref, xs_ref, h_ref, dis_ref, b_ref, gam_ref, bet_ref,
                   wn_ref, h_out, xs_out):
    hn = _bn_block(sc_ref, xs_ref, h_ref, dis_ref, b_ref, gam_ref, bet_ref)
    h_out[...] = hn
    xs_out[...] = jnp.dot(hn, wn_ref[...], precision=_HIGH,
                          preferred_element_type=jnp.float32) * dis_ref[...]


_tc_layer = pl.pallas_call(
    _tc_layer_body,
    out_shape=(
        jax.ShapeDtypeStruct((NPAD, HID), jnp.float32),
        jax.ShapeDtypeStruct((NPAD, HID), jnp.float32),
    ),
    compiler_params=_TC_PARAMS,
)


def _tc_final_body(sc_ref, xs_ref, h_ref, dis_ref, b_ref, gam_ref, bet_ref,
                   batch_ref, w1_ref, b1_ref, w2_ref, b2_ref, w3_ref, b3_ref,
                   out_ref):
    hn = _bn_block(sc_ref, xs_ref, h_ref, dis_ref, b_ref, gam_ref, bet_ref)
    gid = lax.broadcasted_iota(jnp.int32, (N_GRAPHS, NPAD), 0)
    a = (gid == batch_ref[...]).astype(jnp.float32)
    sums = jnp.dot(a, hn, precision=_HIGH, preferred_element_type=jnp.float32)
    cnt = jnp.sum(a, axis=1, keepdims=True)
    g = sums / jnp.maximum(cnt, 1.0)
    t = jnp.maximum(jnp.dot(g, w1_ref[...], precision=_HIGH,
                            preferred_element_type=jnp.float32) + b1_ref[...], 0.0)
    t = jnp.maximum(jnp.dot(t, w2_ref[...], precision=_HIGH,
                            preferred_element_type=jnp.float32) + b2_ref[...], 0.0)
    out_ref[...] = jnp.dot(t, w3_ref[...], precision=_HIGH,
                           preferred_element_type=jnp.float32) + b3_ref[...]


_tc_final = pl.pallas_call(
    _tc_final_body,
    out_shape=jax.ShapeDtypeStruct((N_GRAPHS, N_PROPS), jnp.float32),
    compiler_params=_TC_PARAMS,
)


def kernel(x, params, edge_index, batch):
    f32 = jnp.float32
    pad_idx = (N_NODES
               + jnp.arange(EPAD - N_EDGES, dtype=jnp.int32)
               % (NPAD - N_NODES))
    src_flat = jnp.concatenate([edge_index[0], pad_idx])
    dst_flat = jnp.concatenate([edge_index[1], pad_idx])
    dst_deg = dst_flat.reshape(NW, NCHUNK, CW)
    src = src_flat.reshape(NW, NCHUNK, CW)
    dst = dst_flat.reshape(NW, NCHUNK, CW)
    xp = jnp.pad(x, ((0, NPAD - N_NODES), (0, 0)))
    batch_row = jnp.pad(batch, (0, NPAD - N_NODES),
                        constant_values=N_GRAPHS).reshape(1, NPAD)

    hw1 = params["head_w1"]
    hw2 = params["head_w2"]
    hw3 = params["head_w3"]
    h2, h4 = HID // 2, HID // 4
    w1cat = jnp.concatenate([hw1[p] for p in range(N_PROPS)], axis=1)
    b1cat = jnp.concatenate([params["head_b1"][p] for p in range(N_PROPS)],
                            axis=0).reshape(1, N_PROPS * h2)
    w2blk = jnp.zeros((N_PROPS * h2, N_PROPS * h4), f32)
    for p in range(N_PROPS):
        w2blk = w2blk.at[p * h2:(p + 1) * h2, p * h4:(p + 1) * h4].set(hw2[p])
    b2cat = jnp.concatenate([params["head_b2"][p] for p in range(N_PROPS)],
                            axis=0).reshape(1, N_PROPS * h4)
    w3blk = jnp.zeros((N_PROPS * h4, N_PROPS), f32)
    for p in range(N_PROPS):
        w3blk = w3blk.at[p * h4:(p + 1) * h4, p].set(hw3[p][:, 0])
    b3cat = params["head_b3"].reshape(1, N_PROPS)

    degp = _sc_degree(dst_deg).reshape(NC, NPAD, 1)
    h, xs, dis = _tc_embed(xp, params["node_emb_w"],
                           params["node_emb_b"].reshape(1, HID), degp,
                           params["gcn_w"][0])
    for i in range(N_LAYERS):
        sc_out = _sc_aggregate(xs, src, dst)
        gcn_b = params["gcn_b"][i].reshape(1, HID)
        gam = params["bn_gamma"][i].reshape(1, HID)
        bet = params["bn_beta"][i].reshape(1, HID)
        if i < N_LAYERS - 1:
            h, xs = _tc_layer(sc_out, xs, h, dis, gcn_b, gam, bet,
                              params["gcn_w"][i + 1])
        else:
            preds = _tc_final(sc_out, xs, h, dis, gcn_b, gam, bet, batch_row,
                              w1cat, b1cat, w2blk, b2cat, w3blk, b3cat)
    return preds


# speedup vs baseline: 1.2486x; 1.2486x over previous
import functools

import jax
import jax.numpy as jnp
from jax import lax
from jax.experimental import pallas as pl
from jax.experimental.pallas import tpu as pltpu
from jax.experimental.pallas import tpu_sc as plsc

N_NODES = 10000
N_EDGES = 320000
DIM = 128
HID = 128
N_LAYERS = 4
N_PROPS = 4
N_GRAPHS = 128

NC = 2
NS = 16
NW = NC * NS
CW = 128
NCHUNK = 80
EPW = NCHUNK * CW
EPAD = NW * EPW
NPAD = 10240
RPT = NPAD // NS

_HIGH = lax.Precision.HIGHEST
_TC_PARAMS = pltpu.CompilerParams(vmem_limit_bytes=120 * 1024 * 1024)
_mesh = plsc.VectorSubcoreMesh(core_axis_name="c", subcore_axis_name="s")


@functools.partial(
    pl.kernel,
    mesh=_mesh,
    out_type=jax.ShapeDtypeStruct((NC, NS, RPT), jnp.float32),
    scratch_types=[
        pltpu.VMEM((NCHUNK, CW), jnp.int32),
        pltpu.VMEM((CW,), jnp.float32),
        pltpu.VMEM((RPT,), jnp.float32),
        pltpu.VMEM_SHARED((NPAD,), jnp.float32),
    ],
)
def _sc_degree(dst_hbm, out_hbm, dst_v, ones_v, zrow_v, deg_sh):
    c = lax.axis_index("c")
    s = lax.axis_index("s")
    wid = s * NC + c
    zeros16 = jnp.zeros((16,), jnp.float32)
    ones16 = jnp.ones((16,), jnp.float32)
    for cc in range(CW // 16):
        ones_v[pl.ds(cc * 16, 16)] = ones16

    def zero_local(i, carry):
        zrow_v[pl.ds(i * 16, 16)] = zeros16
        return carry

    lax.fori_loop(0, RPT // 16, zero_local, 0)
    pltpu.sync_copy(zrow_v, deg_sh.at[pl.ds(s * RPT, RPT)])
    pltpu.sync_copy(dst_hbm.at[wid], dst_v)
    plsc.subcore_barrier()

    def count(j, carry):
        pltpu.sync_copy(ones_v, deg_sh.at[dst_v.at[j]], add=True)
        return carry

    lax.fori_loop(0, NCHUNK, count, 0)
    plsc.subcore_barrier()
    pltpu.sync_copy(deg_sh.at[pl.ds(s * RPT, RPT)], out_hbm.at[c, s])


@functools.partial(
    pl.kernel,
    mesh=_mesh,
    out_type=jax.ShapeDtypeStruct((NC, NPAD, HID), jnp.float32),
    scratch_types=[
        pltpu.VMEM((NCHUNK // 2, CW), jnp.int32),
        pltpu.VMEM((NCHUNK // 2, CW), jnp.int32),
        pltpu.VMEM((2, CW, HID), jnp.float32),
        pltpu.VMEM((16, HID), jnp.float32),
        pltpu.VMEM_SHARED((NPAD, HID), jnp.float32),
        pltpu.SemaphoreType.DMA,
        pltpu.SemaphoreType.DMA,
    ],
)
def _sc_aggregate(xs_hbm, src_hbm, dst_hbm, out_hbm,
                  src_v, dst_v, rows_v, zbuf, acc_sh, gsem0, gsem1):
    c = lax.axis_index("c")
    s = lax.axis_index("s")
    wid = s * NC + c
    zeros16 = jnp.zeros((16,), jnp.float32)
    seg_n = NCHUNK // 2

    def zb_row(r, carry):
        for cc in range(HID // 16):
            zbuf[r, pl.ds(cc * 16, 16)] = zeros16
        return carry

    lax.fori_loop(0, 16, zb_row, 0)

    for seg in range(2):
        pltpu.sync_copy(src_hbm.at[wid, pl.ds(seg * seg_n, seg_n)], src_v)
        pltpu.sync_copy(dst_hbm.at[wid, pl.ds(seg * seg_n, seg_n)], dst_v)
        pltpu.async_copy(xs_hbm.at[src_v.at[0]], rows_v.at[0], gsem0)
        pltpu.async_copy(xs_hbm.at[src_v.at[1]], rows_v.at[1], gsem1)
        if seg == 0:
            def zero_slice(i, carry):
                pltpu.sync_copy(zbuf, acc_sh.at[pl.ds(s * RPT + i * 16, 16)])
                return carry

            lax.fori_loop(0, RPT // 16, zero_slice, 0)
            plsc.subcore_barrier()

        def pair(jj, carry):
            j0 = jj * 2
            pltpu.make_async_copy(xs_hbm.at[src_v.at[j0]],
                                  rows_v.at[0], gsem0).wait()
            pltpu.sync_copy(rows_v.at[0], acc_sh.at[dst_v.at[j0]], add=True)

            @pl.when(j0 + 2 < seg_n)
            def _():
                pltpu.async_copy(xs_hbm.at[src_v.at[j0 + 2]],
                                 rows_v.at[0], gsem0)

            pltpu.make_async_copy(xs_hbm.at[src_v.at[j0 + 1]],
                                  rows_v.at[1], gsem1).wait()
            pltpu.sync_copy(rows_v.at[1], acc_sh.at[dst_v.at[j0 + 1]], add=True)

            @pl.when(j0 + 3 < seg_n)
            def _():
                pltpu.async_copy(xs_hbm.at[src_v.at[j0 + 3]],
                                 rows_v.at[1], gsem1)

            return carry

        lax.fori_loop(0, seg_n // 2, pair, 0)

    plsc.subcore_barrier()
    pltpu.sync_copy(acc_sh.at[pl.ds(s * RPT, RPT)],
                    out_hbm.at[c, pl.ds(s * RPT, RPT)])


def _row_mask():
    rowid = lax.broadcasted_iota(jnp.int32, (NPAD, 1), 0)
    return (rowid < N_NODES).astype(jnp.float32)


def _tc_embed_body(x_ref, ew_ref, eb_ref, degp_ref, w0_ref,
                   h_ref, xs_ref, dis_ref):
    h = jnp.maximum(
        jnp.dot(x_ref[...], ew_ref[...], precision=_HIGH,
                preferred_element_type=jnp.float32) + eb_ref[...], 0.0)
    deg = degp_ref[0] + degp_ref[1] + 1.0
    dis = lax.rsqrt(deg) * _row_mask()
    h_ref[...] = h
    dis_ref[...] = dis
    xs_ref[...] = jnp.dot(h, w0_ref[...], precision=_HIGH,
                          preferred_element_type=jnp.float32) * dis


_tc_embed = pl.pallas_call(
    _tc_embed_body,
    out_shape=(
        jax.ShapeDtypeStruct((NPAD, HID), jnp.float32),
        jax.ShapeDtypeStruct((NPAD, HID), jnp.float32),
        jax.ShapeDtypeStruct((NPAD, 1), jnp.float32),
    ),
    compiler_params=_TC_PARAMS,
)


def _bn_block(sc_ref, xs_ref, h_ref, dis_ref, b_ref, gam_ref, bet_ref):
    mask = _row_mask()
    agg = ((sc_ref[0] + sc_ref[1] + xs_ref[...]) * dis_ref[...]
           + b_ref[...]) * mask
    mean = jnp.sum(agg, axis=0, keepdims=True) * (1.0 / N_NODES)
    d = (agg - mean) * mask
    var = jnp.sum(d * d, axis=0, keepdims=True) * (1.0 / N_NODES)
    hbn = (agg - mean) * lax.rsqrt(var + 1e-5) * gam_ref[...] + bet_ref[...]
    return jnp.maximum(hbn, 0.0) + h_ref[...]


def _tc_layer_body(sc_ref, xs_ref, h_ref, dis_ref, b_ref, gam_ref, bet_ref,
                   wn_ref, h_out, xs_out):
    hn = _bn_block(sc_ref, xs_ref, h_ref, dis_ref, b_ref, gam_ref, bet_ref)
    h_out[...] = hn
    xs_out[...] = jnp.dot(hn, wn_ref[...], precision=_HIGH,
                          preferred_element_type=jnp.float32) * dis_ref[...]


_tc_layer = pl.pallas_call(
    _tc_layer_body,
    out_shape=(
        jax.ShapeDtypeStruct((NPAD, HID), jnp.float32),
        jax.ShapeDtypeStruct((NPAD, HID), jnp.float32),
    ),
    compiler_params=_TC_PARAMS,
)


def _tc_final_body(sc_ref, xs_ref, h_ref, dis_ref, b_ref, gam_ref, bet_ref,
                   batch_ref, w1_ref, b1_ref, w2_ref, b2_ref, w3_ref, b3_ref,
                   out_ref):
    hn = _bn_block(sc_ref, xs_ref, h_ref, dis_ref, b_ref, gam_ref, bet_ref)
    gid = lax.broadcasted_iota(jnp.int32, (N_GRAPHS, NPAD), 0)
    a = (gid == batch_ref[...]).astype(jnp.float32)
    sums = jnp.dot(a, hn, precision=_HIGH, preferred_element_type=jnp.float32)
    cnt = jnp.sum(a, axis=1, keepdims=True)
    g = sums / jnp.maximum(cnt, 1.0)
    t = jnp.maximum(jnp.dot(g, w1_ref[...], precision=_HIGH,
                            preferred_element_type=jnp.float32) + b1_ref[...], 0.0)
    t = jnp.maximum(jnp.dot(t, w2_ref[...], precision=_HIGH,
                            preferred_element_type=jnp.float32) + b2_ref[...], 0.0)
    out_ref[...] = jnp.dot(t, w3_ref[...], precision=_HIGH,
                           preferred_element_type=jnp.float32) + b3_ref[...]


_tc_final = pl.pallas_call(
    _tc_final_body,
    out_shape=jax.ShapeDtypeStruct((N_GRAPHS, N_PROPS), jnp.float32),
    compiler_params=_TC_PARAMS,
)


def kernel(x, params, edge_index, batch):
    f32 = jnp.float32
    pad_idx = (N_NODES
               + jnp.arange(EPAD - N_EDGES, dtype=jnp.int32)
               % (NPAD - N_NODES))
    src_flat = jnp.concatenate([edge_index[0], pad_idx])
    dst_flat = jnp.concatenate([edge_index[1], pad_idx])
    dst_deg = dst_flat.reshape(NW, NCHUNK, CW)
    src = src_flat.reshape(NW, NCHUNK, CW)
    dst = dst_flat.reshape(NW, NCHUNK, CW)
    xp = jnp.pad(x, ((0, NPAD - N_NODES), (0, 0)))
    batch_row = jnp.pad(batch, (0, NPAD - N_NODES),
                        constant_values=N_GRAPHS).reshape(1, NPAD)

    hw1 = params["head_w1"]
    hw2 = params["head_w2"]
    hw3 = params["head_w3"]
    h2, h4 = HID // 2, HID // 4
    w1cat = jnp.concatenate([hw1[p] for p in range(N_PROPS)], axis=1)
    b1cat = jnp.concatenate([params["head_b1"][p] for p in range(N_PROPS)],
                            axis=0).reshape(1, N_PROPS * h2)
    w2blk = jnp.zeros((N_PROPS * h2, N_PROPS * h4), f32)
    for p in range(N_PROPS):
        w2blk = w2blk.at[p * h2:(p + 1) * h2, p * h4:(p + 1) * h4].set(hw2[p])
    b2cat = jnp.concatenate([params["head_b2"][p] for p in range(N_PROPS)],
                            axis=0).reshape(1, N_PROPS * h4)
    w3blk = jnp.zeros((N_PROPS * h4, N_PROPS), f32)
    for p in range(N_PROPS):
        w3blk = w3blk.at[p * h4:(p + 1) * h4, p].set(hw3[p][:, 0])
    b3cat = params["head_b3"].reshape(1, N_PROPS)

    degp = _sc_degree(dst_deg).reshape(NC, NPAD, 1)
    h, xs, dis = _tc_embed(xp, params["node_emb_w"],
                           params["node_emb_b"].reshape(1, HID), degp,
                           params["gcn_w"][0])
    for i in range(N_LAYERS):
        sc_out = _sc_aggregate(xs, src, dst)
        gcn_b = params["gcn_b"][i].reshape(1, HID)
        gam = params["bn_gamma"][i].reshape(1, HID)
        bet = params["bn_beta"][i].reshape(1, HID)
        if i < N_LAYERS - 1:
            h, xs = _tc_layer(sc_out, xs, h, dis, gcn_b, gam, bet,
                              params["gcn_w"][i + 1])
        else:
            preds = _tc_final(sc_out, xs, h, dis, gcn_b, gam, bet, batch_row,
                              w1cat, b1cat, w2blk, b2cat, w3blk, b3cat)
    return preds


# speedup vs baseline: 1.2578x; 1.0074x over previous
import functools

import jax
import jax.numpy as jnp
from jax import lax
from jax.experimental import pallas as pl
from jax.experimental.pallas import tpu as pltpu
from jax.experimental.pallas import tpu_sc as plsc

N_NODES = 10000
N_EDGES = 320000
DIM = 128
HID = 128
N_LAYERS = 4
N_PROPS = 4
N_GRAPHS = 128

NC = 2
NS = 16
NW = NC * NS
CW = 128
NCHUNK = 80
EPW = NCHUNK * CW
EPAD = NW * EPW
NPAD = 10240
RPT = NPAD // NS

_HIGH = lax.Precision.HIGHEST
_TC_PARAMS = pltpu.CompilerParams(vmem_limit_bytes=120 * 1024 * 1024)
_mesh = plsc.VectorSubcoreMesh(core_axis_name="c", subcore_axis_name="s")


@functools.partial(
    pl.kernel,
    mesh=_mesh,
    out_type=jax.ShapeDtypeStruct((NC, NS, RPT), jnp.float32),
    scratch_types=[
        pltpu.VMEM((NCHUNK, CW), jnp.int32),
        pltpu.VMEM((CW,), jnp.float32),
        pltpu.VMEM((RPT,), jnp.float32),
        pltpu.VMEM_SHARED((NPAD,), jnp.float32),
    ],
)
def _sc_degree(dst_hbm, out_hbm, dst_v, ones_v, zrow_v, deg_sh):
    c = lax.axis_index("c")
    s = lax.axis_index("s")
    wid = s * NC + c
    zeros16 = jnp.zeros((16,), jnp.float32)
    ones16 = jnp.ones((16,), jnp.float32)
    for cc in range(CW // 16):
        ones_v[pl.ds(cc * 16, 16)] = ones16

    def zero_local(i, carry):
        zrow_v[pl.ds(i * 16, 16)] = zeros16
        return carry

    lax.fori_loop(0, RPT // 16, zero_local, 0)
    pltpu.sync_copy(zrow_v, deg_sh.at[pl.ds(s * RPT, RPT)])
    pltpu.sync_copy(dst_hbm.at[wid], dst_v)
    plsc.subcore_barrier()

    def count(j, carry):
        pltpu.sync_copy(ones_v, deg_sh.at[dst_v.at[j]], add=True)
        return carry

    lax.fori_loop(0, NCHUNK, count, 0)
    plsc.subcore_barrier()
    pltpu.sync_copy(deg_sh.at[pl.ds(s * RPT, RPT)], out_hbm.at[c, s])


@functools.partial(
    pl.kernel,
    mesh=_mesh,
    out_type=jax.ShapeDtypeStruct((NC, NPAD, HID), jnp.float32),
    scratch_types=[
        pltpu.VMEM((NCHUNK // 2, CW), jnp.int32),
        pltpu.VMEM((NCHUNK // 2, CW), jnp.int32),
        pltpu.VMEM((2, CW, HID), jnp.float32),
        pltpu.VMEM((16, HID), jnp.float32),
        pltpu.VMEM_SHARED((NPAD, HID), jnp.float32),
        pltpu.SemaphoreType.DMA,
        pltpu.SemaphoreType.DMA,
    ],
)
def _sc_aggregate(xs_hbm, src_hbm, dst_hbm, out_hbm,
                  src_v, dst_v, rows_v, zbuf, acc_sh, gsem0, gsem1):
    c = lax.axis_index("c")
    s = lax.axis_index("s")
    wid = s * NC + c
    zeros16 = jnp.zeros((16,), jnp.float32)
    seg_n = NCHUNK // 2

    def zb_row(r, carry):
        for cc in range(HID // 16):
            zbuf[r, pl.ds(cc * 16, 16)] = zeros16
        return carry

    lax.fori_loop(0, 16, zb_row, 0)

    for seg in range(2):
        pltpu.sync_copy(src_hbm.at[wid, pl.ds(seg * seg_n, seg_n)], src_v)
        pltpu.sync_copy(dst_hbm.at[wid, pl.ds(seg * seg_n, seg_n)], dst_v)
        pltpu.async_copy(xs_hbm.at[src_v.at[0]], rows_v.at[0], gsem0)
        pltpu.async_copy(xs_hbm.at[src_v.at[1]], rows_v.at[1], gsem1)
        if seg == 0:
            def zero_slice(i, carry):
                pltpu.sync_copy(zbuf, acc_sh.at[pl.ds(s * RPT + i * 16, 16)])
                return carry

            lax.fori_loop(0, RPT // 16, zero_slice, 0)
            plsc.subcore_barrier()

        def pair(jj, carry):
            j0 = jj * 2
            pltpu.make_async_copy(xs_hbm.at[src_v.at[j0]],
                                  rows_v.at[0], gsem0).wait()
            pltpu.sync_copy(rows_v.at[0], acc_sh.at[dst_v.at[j0]], add=True)

            @pl.when(j0 + 2 < seg_n)
            def _():
                pltpu.async_copy(xs_hbm.at[src_v.at[j0 + 2]],
                                 rows_v.at[0], gsem0)

            pltpu.make_async_copy(xs_hbm.at[src_v.at[j0 + 1]],
                                  rows_v.at[1], gsem1).wait()
            pltpu.sync_copy(rows_v.at[1], acc_sh.at[dst_v.at[j0 + 1]], add=True)

            @pl.when(j0 + 3 < seg_n)
            def _():
                pltpu.async_copy(xs_hbm.at[src_v.at[j0 + 3]],
                                 rows_v.at[1], gsem1)

            return carry

        lax.fori_loop(0, seg_n // 2, pair, 0)

    plsc.subcore_barrier()
    pltpu.sync_copy(acc_sh.at[pl.ds(s * RPT, RPT)],
                    out_hbm.at[c, pl.ds(s * RPT, RPT)])


def _row_mask():
    rowid = lax.broadcasted_iota(jnp.int32, (NPAD, 1), 0)
    return (rowid < N_NODES).astype(jnp.float32)


def _tc_embed_body(x_ref, ew_ref, eb_ref, degp_ref, w0_ref,
                   h_ref, xs_ref, dis_ref):
    h = jnp.maximum(
        jnp.dot(x_ref[...], ew_ref[...], precision=_HIGH,
                preferred_element_type=jnp.float32) + eb_ref[...], 0.0)
    deg = degp_ref[0] + degp_ref[1] + 1.0
    dis = lax.rsqrt(deg) * _row_mask()
    h_ref[...] = h
    dis_ref[...] = jnp.broadcast_to(dis, (NPAD, HID))
    xs_ref[...] = jnp.dot(h, w0_ref[...], precision=_HIGH,
                          preferred_element_type=jnp.float32) * dis


_tc_embed = pl.pallas_call(
    _tc_embed_body,
    out_shape=(
        jax.ShapeDtypeStruct((NPAD, HID), jnp.float32),
        jax.ShapeDtypeStruct((NPAD, HID), jnp.float32),
        jax.ShapeDtypeStruct((NPAD, HID), jnp.float32),
    ),
    compiler_params=_TC_PARAMS,
)


def _bn_block(sc_ref, xs_ref, h_ref, dis_ref, b_ref, gam_ref, bet_ref):
    mask = _row_mask()
    agg = ((sc_ref[0] + sc_ref[1] + xs_ref[...]) * dis_ref[...]
           + b_ref[...]) * mask
    mean = jnp.sum(agg, axis=0, keepdims=True) * (1.0 / N_NODES)
    d = (agg - mean) * mask
    var = jnp.sum(d * d, axis=0, keepdims=True) * (1.0 / N_NODES)
    hbn = (agg - mean) * lax.rsqrt(var + 1e-5) * gam_ref[...] + bet_ref[...]
    return jnp.maximum(hbn, 0.0) + h_ref[...]


def _tc_layer_body(sc_ref, xs_ref, h_ref, dis_ref, b_ref, gam_ref, bet_ref,
                   wn_ref, h_out, xs_out):
    hn = _bn_block(sc_ref, xs_ref, h_ref, dis_ref, b_ref, gam_ref, bet_ref)
    h_out[...] = hn
    xs_out[...] = jnp.dot(hn, wn_ref[...], precision=_HIGH,
                          preferred_element_type=jnp.float32) * dis_ref[...]


_tc_layer = pl.pallas_call(
    _tc_layer_body,
    out_shape=(
        jax.ShapeDtypeStruct((NPAD, HID), jnp.float32),
        jax.ShapeDtypeStruct((NPAD, HID), jnp.float32),
    ),
    compiler_params=_TC_PARAMS,
)


def _tc_final_body(sc_ref, xs_ref, h_ref, dis_ref, b_ref, gam_ref, bet_ref,
                   batch_ref, w1_ref, b1_ref, w2_ref, b2_ref, w3_ref, b3_ref,
                   out_ref):
    hn = _bn_block(sc_ref, xs_ref, h_ref, dis_ref, b_ref, gam_ref, bet_ref)
    gid = lax.broadcasted_iota(jnp.int32, (N_GRAPHS, NPAD), 0)
    a = (gid == batch_ref[...]).astype(jnp.float32)
    sums = jnp.dot(a, hn, precision=_HIGH, preferred_element_type=jnp.float32)
    cnt = jnp.sum(a, axis=1, keepdims=True)
    g = sums / jnp.maximum(cnt, 1.0)
    t = jnp.maximum(jnp.dot(g, w1_ref[...], precision=_HIGH,
                            preferred_element_type=jnp.float32) + b1_ref[...], 0.0)
    t = jnp.maximum(jnp.dot(t, w2_ref[...], precision=_HIGH,
                            preferred_element_type=jnp.float32) + b2_ref[...], 0.0)
    out_ref[...] = jnp.dot(t, w3_ref[...], precision=_HIGH,
                           preferred_element_type=jnp.float32) + b3_ref[...]


_tc_final = pl.pallas_call(
    _tc_final_body,
    out_shape=jax.ShapeDtypeStruct((N_GRAPHS, N_PROPS), jnp.float32),
    compiler_params=_TC_PARAMS,
)


def kernel(x, params, edge_index, batch):
    f32 = jnp.float32
    pad_idx = (N_NODES
               + jnp.arange(EPAD - N_EDGES, dtype=jnp.int32)
               % (NPAD - N_NODES))
    src_flat = jnp.concatenate([edge_index[0], pad_idx])
    dst_flat = jnp.concatenate([edge_index[1], pad_idx])
    dst_deg = dst_flat.reshape(NW, NCHUNK, CW)
    src = src_flat.reshape(NW, NCHUNK, CW)
    dst = dst_flat.reshape(NW, NCHUNK, CW)
    xp = jnp.pad(x, ((0, NPAD - N_NODES), (0, 0)))
    batch_row = jnp.pad(batch, (0, NPAD - N_NODES),
                        constant_values=N_GRAPHS).reshape(1, NPAD)

    hw1 = params["head_w1"]
    hw2 = params["head_w2"]
    hw3 = params["head_w3"]
    h2, h4 = HID // 2, HID // 4
    w1cat = jnp.concatenate([hw1[p] for p in range(N_PROPS)], axis=1)
    b1cat = jnp.concatenate([params["head_b1"][p] for p in range(N_PROPS)],
                            axis=0).reshape(1, N_PROPS * h2)
    w2blk = jnp.zeros((N_PROPS * h2, N_PROPS * h4), f32)
    for p in range(N_PROPS):
        w2blk = w2blk.at[p * h2:(p + 1) * h2, p * h4:(p + 1) * h4].set(hw2[p])
    b2cat = jnp.concatenate([params["head_b2"][p] for p in range(N_PROPS)],
                            axis=0).reshape(1, N_PROPS * h4)
    w3blk = jnp.zeros((N_PROPS * h4, N_PROPS), f32)
    for p in range(N_PROPS):
        w3blk = w3blk.at[p * h4:(p + 1) * h4, p].set(hw3[p][:, 0])
    b3cat = params["head_b3"].reshape(1, N_PROPS)

    degp = _sc_degree(dst_deg).reshape(NC, NPAD, 1)
    h, xs, dis = _tc_embed(xp, params["node_emb_w"],
                           params["node_emb_b"].reshape(1, HID), degp,
                           params["gcn_w"][0])
    for i in range(N_LAYERS):
        sc_out = _sc_aggregate(xs, src, dst)
        gcn_b = params["gcn_b"][i].reshape(1, HID)
        gam = params["bn_gamma"][i].reshape(1, HID)
        bet = params["bn_beta"][i].reshape(1, HID)
        if i < N_LAYERS - 1:
            h, xs = _tc_layer(sc_out, xs, h, dis, gcn_b, gam, bet,
                              params["gcn_w"][i + 1])
        else:
            preds = _tc_final(sc_out, xs, h, dis, gcn_b, gam, bet, batch_row,
                              w1cat, b1cat, w2blk, b2cat, w3blk, b3cat)
    return preds
